# baseline (speedup 1.0000x reference)
import functools

import jax
import jax.numpy as jnp
from jax import lax
from jax.experimental import pallas as pl
from jax.experimental.pallas import tpu as pltpu
from jax.experimental.pallas import tpu_sc as plsc

NC = 2
NS = 16
CH = 128
TL = 16
DW = 16
CPB = 6
IBW = CPB * CH


def _sc_segment_sum(data, structure, n_pad, d):
    n, e = data.shape[0], structure.shape[1]
    del n
    edges_per_tile = e // (NC * NS)
    chunks = edges_per_tile // CH
    assert chunks * CH + TL == edges_per_tile and chunks % 2 == 0
    nblk = chunks // CPB
    assert nblk * CPB == chunks and nblk % 2 == 1
    ppb = CPB // 2
    rows_per_tile = n_pad // NS
    mesh = plsc.VectorSubcoreMesh(core_axis_name="c", subcore_axis_name="s")

    @functools.partial(
        pl.kernel,
        mesh=mesh,
        compiler_params=pltpu.CompilerParams(use_tc_tiling_on_sc=False),
        out_type=[
            jax.ShapeDtypeStruct((NC, n_pad, d), jnp.float32),
            jax.ShapeDtypeStruct((NC, n_pad, DW), jnp.float32),
        ],
        scratch_types=[
            pltpu.VMEM((2, IBW), jnp.int32),
            pltpu.VMEM((2, IBW), jnp.int32),
            pltpu.VMEM((CH, d), jnp.float32),
            pltpu.VMEM((CH, d), jnp.float32),
            pltpu.VMEM((CH, DW), jnp.float32),
            pltpu.VMEM_SHARED((n_pad, d), jnp.float32),
            pltpu.VMEM_SHARED((n_pad, DW), jnp.float32),
            pltpu.SemaphoreType.DMA,
            pltpu.SemaphoreType.DMA,
            pltpu.SemaphoreType.DMA,
            pltpu.SemaphoreType.DMA,
            pltpu.SemaphoreType.DMA,
        ],
    )
    def sc_kernel(data_hbm, struct_hbm, out_hbm, deg_hbm,
                  idxa, idxb, rows0, rows1, ones_v,
                  out_sh, deg_sh, sem0, sem1, semi, ssem0, ssem1):
        cid = lax.axis_index("c")
        sid = lax.axis_index("s")
        wid = cid * NS + sid
        ebase = wid * edges_per_tile
        rbase = sid * rows_per_tile

        def blk_refs(buf, b):
            return [(struct_hbm.at[r, pl.ds(ebase + b * IBW, IBW)], buf.at[r])
                    for r in range(2)]

        def fetch_blk(buf, b):
            for s, t in blk_refs(buf, b):
                pltpu.async_copy(s, t, semi)

        def wait_blk(buf, b):
            for s, t in blk_refs(buf, b):
                pltpu.make_async_copy(s, t, semi).wait()

        fetch_blk(idxa, 0)

        @pl.loop(0, CH)
        def _(i):
            ones_v.at[pl.ds(i, 1), :][...] = jnp.ones((1, DW), jnp.float32)

            @pl.loop(0, d, step=DW)
            def _(c):
                rows0.at[pl.ds(i, 1), pl.ds(c, DW)][...] = (
                    jnp.zeros((1, DW), jnp.float32))

        zrow = rows0.at[pl.ds(0, CH), pl.ds(0, DW)]
        zrow_t = rows0.at[pl.ds(0, 120), pl.ds(0, DW)]

        @pl.loop(0, 4)
        def _(i):
            pltpu.async_copy(rows0, out_sh.at[pl.ds(rbase + i * CH, CH)], semi)
            pltpu.async_copy(zrow, deg_sh.at[pl.ds(rbase + i * CH, CH)], semi)
        pltpu.async_copy(rows0.at[pl.ds(0, 120)],
                         out_sh.at[pl.ds(rbase + 4 * CH, 120)], semi)
        pltpu.async_copy(zrow_t, deg_sh.at[pl.ds(rbase + 4 * CH, 120)], semi)

        @pl.loop(0, 4)
        def _(i):
            pltpu.make_async_copy(
                rows0, out_sh.at[pl.ds(rbase + i * CH, CH)], semi).wait()
            pltpu.make_async_copy(
                zrow, deg_sh.at[pl.ds(rbase + i * CH, CH)], semi).wait()
        pltpu.make_async_copy(
            rows0.at[pl.ds(0, 120)],
            out_sh.at[pl.ds(rbase + 4 * CH, 120)], semi).wait()
        pltpu.make_async_copy(
            zrow_t, deg_sh.at[pl.ds(rbase + 4 * CH, 120)], semi).wait()
        wait_blk(idxa, 0)
        plsc.subcore_barrier()

        HH = CH // 2

        def gather(buf, l, rows, sem):
            for h in range(2):
                pltpu.async_copy(
                    data_hbm.at[buf.at[0, pl.ds(l * CH + h * HH, HH)]],
                    rows.at[pl.ds(h * HH, HH)], sem)

        def wait_gather(buf, l, rows, sem):
            for h in range(2):
                pltpu.make_async_copy(
                    data_hbm.at[buf.at[0, pl.ds(l * CH + h * HH, HH)]],
                    rows.at[pl.ds(h * HH, HH)], sem).wait()

        def scatter(buf, l, rows, ssem):
            for h in range(2):
                pltpu.async_copy(
                    rows.at[pl.ds(h * HH, HH)],
                    out_sh.at[buf.at[1, pl.ds(l * CH + h * HH, HH)]],
                    ssem, add=True)
                pltpu.async_copy(
                    ones_v.at[pl.ds(h * HH, HH)],
                    deg_sh.at[buf.at[1, pl.ds(l * CH + h * HH, HH)]],
                    ssem, add=True)

        def wait_scatter(buf, l, rows, ssem):
            for h in range(2):
                pltpu.make_async_copy(
                    rows.at[pl.ds(h * HH, HH)],
                    out_sh.at[buf.at[1, pl.ds(l * CH + h * HH, HH)]],
                    ssem).wait()
                pltpu.make_async_copy(
                    ones_v.at[pl.ds(h * HH, HH)],
                    deg_sh.at[buf.at[1, pl.ds(l * CH + h * HH, HH)]],
                    ssem).wait()

        gather(idxa, 0, rows0, sem0)

        def do_block(b, cur, nxt, last):
            @pl.loop(0, ppb)
            def _(p):
                l0 = 2 * p
                l1 = l0 + 1
                wait_gather(cur, l0, rows0, sem0)

                @pl.when(p == 0)
                def _():
                    @pl.when(b > 0)
                    def _():
                        wait_scatter(nxt, CPB - 1, rows1, ssem1)
                    if not last:
                        fetch_blk(nxt, b + 1)

                @pl.when(p > 0)
                def _():
                    wait_scatter(cur, l0 - 1, rows1, ssem1)

                gather(cur, l1, rows1, sem1)
                scatter(cur, l0, rows0, ssem0)

                wait_gather(cur, l1, rows1, sem1)
                wait_scatter(cur, l0, rows0, ssem0)

                @pl.when(p < ppb - 1)
                def _():
                    gather(cur, l0 + 2, rows0, sem0)

                if not last:
                    @pl.when(p == ppb - 1)
                    def _():
                        wait_blk(nxt, b + 1)
                        gather(nxt, 0, rows0, sem0)

                scatter(cur, l1, rows1, ssem1)

        @pl.loop(0, nblk // 2)
        def _(sb):
            do_block(2 * sb, idxa, idxb, last=False)
            do_block(2 * sb + 1, idxb, idxa, last=False)
        do_block(nblk - 1, idxa, idxb, last=True)

        toff = ebase + chunks * CH
        pltpu.async_copy(struct_hbm.at[0, pl.ds(toff, TL)],
                         idxb.at[0, pl.ds(0, TL)], semi)
        pltpu.async_copy(struct_hbm.at[1, pl.ds(toff, TL)],
                         idxb.at[1, pl.ds(0, TL)], semi)
        pltpu.make_async_copy(struct_hbm.at[0, pl.ds(toff, TL)],
                              idxb.at[0, pl.ds(0, TL)], semi).wait()
        pltpu.make_async_copy(struct_hbm.at[1, pl.ds(toff, TL)],
                              idxb.at[1, pl.ds(0, TL)], semi).wait()
        wait_scatter(idxa, CPB - 1, rows1, ssem1)
        pltpu.async_copy(data_hbm.at[idxb.at[0, pl.ds(0, TL)]],
                         rows1.at[pl.ds(0, TL)], sem0).wait()
        pltpu.sync_copy(rows1.at[pl.ds(0, TL)],
                        out_sh.at[idxb.at[1, pl.ds(0, TL)]], add=True)
        pltpu.sync_copy(ones_v.at[pl.ds(0, TL)],
                        deg_sh.at[idxb.at[1, pl.ds(0, TL)]], add=True)

        plsc.subcore_barrier()
        pltpu.sync_copy(out_sh.at[pl.ds(rbase, rows_per_tile)],
                        out_hbm.at[cid].at[pl.ds(rbase, rows_per_tile)])
        pltpu.sync_copy(deg_sh.at[pl.ds(rbase, rows_per_tile)],
                        deg_hbm.at[cid].at[pl.ds(rbase, rows_per_tile)])

    return sc_kernel(data, structure)


def _mm(data, merge, w1t, b1, w2t, b2, n, d):
    br = 1000

    def body(data_b, merge_b, w1t_b, b1_b, w2t_b, b2_b, o_b, s_b):
        o_b[...] = jnp.dot(data_b[...], w1t_b[...],
                           preferred_element_type=jnp.float32,
                           precision=lax.Precision.HIGHEST) + b1_b[...]
        s_b[...] = jnp.dot(merge_b[...], w2t_b[...],
                           preferred_element_type=jnp.float32,
                           precision=lax.Precision.HIGHEST) + b2_b[...]

    full = lambda shape: pl.BlockSpec(shape, lambda i: tuple(0 for _ in shape))
    return pl.pallas_call(
        body,
        grid=(n // br,),
        in_specs=[
            pl.BlockSpec((br, d), lambda i: (i, 0)),
            pl.BlockSpec((br, d), lambda i: (i, 0)),
            full((d, d)), full((1, d)), full((d, d)), full((1, d)),
        ],
        out_specs=[pl.BlockSpec((br, d), lambda i: (i, 0)),
                   pl.BlockSpec((br, d), lambda i: (i, 0))],
        out_shape=[jax.ShapeDtypeStruct((n, d), jnp.float32),
                   jax.ShapeDtypeStruct((n, d), jnp.float32)],
    )(data, merge, w1t, b1, w2t, b2)


def _combine(out, skip, partials, degs, w1t, b1, n, d):
    br = 1000

    def body(out_b, skip_b, p0_b, p1_b, d0_b, d1_b, w1t_b, b1_b, o_b):
        agg = p0_b[0] + p1_b[0]
        deg = d0_b[0, :, :1] + d1_b[0, :, :1]
        msg = jnp.dot(agg, w1t_b[...],
                      preferred_element_type=jnp.float32,
                      precision=lax.Precision.HIGHEST) + deg * b1_b[...]
        lap = (deg * out_b[...] - msg) / jnp.maximum(deg, 1.0)
        o_b[...] = jnp.maximum(lap + skip_b[...], 0.0)

    full = lambda shape: pl.BlockSpec(shape, lambda i: tuple(0 for _ in shape))
    return pl.pallas_call(
        body,
        grid=(n // br,),
        in_specs=[
            pl.BlockSpec((br, d), lambda i: (i, 0)),
            pl.BlockSpec((br, d), lambda i: (i, 0)),
            pl.BlockSpec((1, br, d), lambda i: (0, i, 0)),
            pl.BlockSpec((1, br, d), lambda i: (1, i, 0)),
            pl.BlockSpec((1, br, DW), lambda i: (0, i, 0)),
            pl.BlockSpec((1, br, DW), lambda i: (1, i, 0)),
            full((d, d)), full((1, d)),
        ],
        out_specs=pl.BlockSpec((br, d), lambda i: (i, 0)),
        out_shape=jax.ShapeDtypeStruct((n, d), jnp.float32),
    )(out, skip, partials, partials, degs, degs, w1t, b1)


def kernel(data, merge, structure, W_lin, b_lin, W_tr, b_tr):
    n, d = data.shape
    n_pad = ((n + 127) // 128) * 128

    w1t = W_lin.T
    w2t = W_tr.T
    out, skip = _mm(data, merge, w1t, b_lin[None, :], w2t, b_tr[None, :], n, d)
    partials, degs = _sc_segment_sum(data, structure, n_pad, d)
    return _combine(out, skip, partials, degs, w1t, b_lin[None, :], n, d)


# speedup vs baseline: 1.0284x; 1.0284x over previous
import functools

import jax
import jax.numpy as jnp
from jax import lax
from jax.experimental import pallas as pl
from jax.experimental.pallas import tpu as pltpu
from jax.experimental.pallas import tpu_sc as plsc

NC = 2
NS = 16
CH = 128
TL = 16
DW = 16
CPB = 6
IBW = CPB * CH


def _sc_segment_sum(data, structure, n_pad, d):
    n, e = data.shape[0], structure.shape[1]
    del n
    edges_per_tile = e // (NC * NS)
    chunks = edges_per_tile // CH
    assert chunks * CH + TL == edges_per_tile and chunks % 2 == 0
    nblk = chunks // CPB
    assert nblk * CPB == chunks and nblk % 2 == 1
    ppb = CPB // 2
    rows_per_tile = n_pad // NS
    mesh = plsc.VectorSubcoreMesh(core_axis_name="c", subcore_axis_name="s")

    @functools.partial(
        pl.kernel,
        mesh=mesh,
        compiler_params=pltpu.CompilerParams(use_tc_tiling_on_sc=False),
        out_type=[
            jax.ShapeDtypeStruct((NC, n_pad, d), jnp.float32),
            jax.ShapeDtypeStruct((NC, n_pad, DW), jnp.float32),
        ],
        scratch_types=[
            pltpu.VMEM((2, IBW), jnp.int32),
            pltpu.VMEM((2, IBW), jnp.int32),
            pltpu.VMEM((CH, d), jnp.float32),
            pltpu.VMEM((CH, d), jnp.float32),
            pltpu.VMEM((CH, DW), jnp.float32),
            pltpu.VMEM_SHARED((n_pad, d), jnp.float32),
            pltpu.VMEM_SHARED((n_pad, DW), jnp.float32),
            pltpu.SemaphoreType.DMA,
            pltpu.SemaphoreType.DMA,
            pltpu.SemaphoreType.DMA,
            pltpu.SemaphoreType.DMA,
            pltpu.SemaphoreType.DMA,
        ],
    )
    def sc_kernel(data_hbm, struct_hbm, out_hbm, deg_hbm,
                  idxa, idxb, rows0, rows1, ones_v,
                  out_sh, deg_sh, sem0, sem1, semi, ssem0, ssem1):
        cid = lax.axis_index("c")
        sid = lax.axis_index("s")
        wid = cid * NS + sid
        ebase = wid * edges_per_tile
        rbase = sid * rows_per_tile

        def blk_refs(buf, b):
            return [(struct_hbm.at[r, pl.ds(ebase + b * IBW, IBW)], buf.at[r])
                    for r in range(2)]

        def fetch_blk(buf, b):
            for s, t in blk_refs(buf, b):
                pltpu.async_copy(s, t, semi)

        def wait_blk(buf, b):
            for s, t in blk_refs(buf, b):
                pltpu.make_async_copy(s, t, semi).wait()

        fetch_blk(idxa, 0)

        @pl.loop(0, CH)
        def _(i):
            ones_v.at[pl.ds(i, 1), :][...] = jnp.ones((1, DW), jnp.float32)

            @pl.loop(0, d, step=DW)
            def _(c):
                rows0.at[pl.ds(i, 1), pl.ds(c, DW)][...] = (
                    jnp.zeros((1, DW), jnp.float32))

        zrow = rows0.at[pl.ds(0, CH), pl.ds(0, DW)]
        zrow_t = rows0.at[pl.ds(0, 120), pl.ds(0, DW)]

        @pl.loop(0, 4)
        def _(i):
            pltpu.async_copy(rows0, out_sh.at[pl.ds(rbase + i * CH, CH)], semi)
            pltpu.async_copy(zrow, deg_sh.at[pl.ds(rbase + i * CH, CH)], semi)
        pltpu.async_copy(rows0.at[pl.ds(0, 120)],
                         out_sh.at[pl.ds(rbase + 4 * CH, 120)], semi)
        pltpu.async_copy(zrow_t, deg_sh.at[pl.ds(rbase + 4 * CH, 120)], semi)

        @pl.loop(0, 4)
        def _(i):
            pltpu.make_async_copy(
                rows0, out_sh.at[pl.ds(rbase + i * CH, CH)], semi).wait()
            pltpu.make_async_copy(
                zrow, deg_sh.at[pl.ds(rbase + i * CH, CH)], semi).wait()
        pltpu.make_async_copy(
            rows0.at[pl.ds(0, 120)],
            out_sh.at[pl.ds(rbase + 4 * CH, 120)], semi).wait()
        pltpu.make_async_copy(
            zrow_t, deg_sh.at[pl.ds(rbase + 4 * CH, 120)], semi).wait()
        wait_blk(idxa, 0)
        plsc.subcore_barrier()

        def gather(buf, l, rows, sem):
            pltpu.async_copy(data_hbm.at[buf.at[0, pl.ds(l * CH, CH)]],
                             rows, sem)

        def wait_gather(buf, l, rows, sem):
            pltpu.make_async_copy(data_hbm.at[buf.at[0, pl.ds(l * CH, CH)]],
                                  rows, sem).wait()

        def scatter(buf, l, rows, ssem):
            pltpu.async_copy(rows, out_sh.at[buf.at[1, pl.ds(l * CH, CH)]],
                             ssem, add=True)
            pltpu.async_copy(ones_v, deg_sh.at[buf.at[1, pl.ds(l * CH, CH)]],
                             ssem, add=True)

        def wait_scatter(buf, l, rows, ssem):
            pltpu.make_async_copy(
                rows, out_sh.at[buf.at[1, pl.ds(l * CH, CH)]], ssem).wait()
            pltpu.make_async_copy(
                ones_v, deg_sh.at[buf.at[1, pl.ds(l * CH, CH)]], ssem).wait()

        gather(idxa, 0, rows0, sem0)

        def do_block(b, cur, nxt, last):
            @pl.loop(0, ppb)
            def _(p):
                l0 = 2 * p
                l1 = l0 + 1
                wait_gather(cur, l0, rows0, sem0)

                @pl.when(p == 0)
                def _():
                    @pl.when(b > 0)
                    def _():
                        wait_scatter(nxt, CPB - 1, rows1, ssem1)
                    if not last:
                        fetch_blk(nxt, b + 1)

                @pl.when(p > 0)
                def _():
                    wait_scatter(cur, l0 - 1, rows1, ssem1)

                gather(cur, l1, rows1, sem1)
                scatter(cur, l0, rows0, ssem0)

                wait_gather(cur, l1, rows1, sem1)
                wait_scatter(cur, l0, rows0, ssem0)

                @pl.when(p < ppb - 1)
                def _():
                    gather(cur, l0 + 2, rows0, sem0)

                if not last:
                    @pl.when(p == ppb - 1)
                    def _():
                        wait_blk(nxt, b + 1)
                        gather(nxt, 0, rows0, sem0)

                scatter(cur, l1, rows1, ssem1)

        @pl.loop(0, nblk // 2)
        def _(sb):
            do_block(2 * sb, idxa, idxb, last=False)
            do_block(2 * sb + 1, idxb, idxa, last=False)
        do_block(nblk - 1, idxa, idxb, last=True)

        toff = ebase + chunks * CH
        pltpu.async_copy(struct_hbm.at[0, pl.ds(toff, TL)],
                         idxb.at[0, pl.ds(0, TL)], semi)
        pltpu.async_copy(struct_hbm.at[1, pl.ds(toff, TL)],
                         idxb.at[1, pl.ds(0, TL)], semi)
        pltpu.make_async_copy(struct_hbm.at[0, pl.ds(toff, TL)],
                              idxb.at[0, pl.ds(0, TL)], semi).wait()
        pltpu.make_async_copy(struct_hbm.at[1, pl.ds(toff, TL)],
                              idxb.at[1, pl.ds(0, TL)], semi).wait()
        wait_scatter(idxa, CPB - 1, rows1, ssem1)
        pltpu.async_copy(data_hbm.at[idxb.at[0, pl.ds(0, TL)]],
                         rows1.at[pl.ds(0, TL)], sem0).wait()
        pltpu.sync_copy(rows1.at[pl.ds(0, TL)],
                        out_sh.at[idxb.at[1, pl.ds(0, TL)]], add=True)
        pltpu.sync_copy(ones_v.at[pl.ds(0, TL)],
                        deg_sh.at[idxb.at[1, pl.ds(0, TL)]], add=True)

        plsc.subcore_barrier()
        pltpu.sync_copy(out_sh.at[pl.ds(rbase, rows_per_tile)],
                        out_hbm.at[cid].at[pl.ds(rbase, rows_per_tile)])
        pltpu.sync_copy(deg_sh.at[pl.ds(rbase, rows_per_tile)],
                        deg_hbm.at[cid].at[pl.ds(rbase, rows_per_tile)])

    return sc_kernel(data, structure)


def _mm(data, merge, w1t, b1, w2t, b2, n, d):
    br = 1000

    def body(data_b, merge_b, w1t_b, b1_b, w2t_b, b2_b, o_b, s_b):
        o_b[...] = jnp.dot(data_b[...], w1t_b[...],
                           preferred_element_type=jnp.float32) + b1_b[...]
        s_b[...] = jnp.dot(merge_b[...], w2t_b[...],
                           preferred_element_type=jnp.float32) + b2_b[...]

    full = lambda shape: pl.BlockSpec(shape, lambda i: tuple(0 for _ in shape))
    return pl.pallas_call(
        body,
        grid=(n // br,),
        in_specs=[
            pl.BlockSpec((br, d), lambda i: (i, 0)),
            pl.BlockSpec((br, d), lambda i: (i, 0)),
            full((d, d)), full((1, d)), full((d, d)), full((1, d)),
        ],
        out_specs=[pl.BlockSpec((br, d), lambda i: (i, 0)),
                   pl.BlockSpec((br, d), lambda i: (i, 0))],
        out_shape=[jax.ShapeDtypeStruct((n, d), jnp.float32),
                   jax.ShapeDtypeStruct((n, d), jnp.float32)],
    )(data, merge, w1t, b1, w2t, b2)


def _combine(out, skip, partials, degs, w1t, b1, n, d):
    br = 1000

    def body(out_b, skip_b, p0_b, p1_b, d0_b, d1_b, w1t_b, b1_b, o_b):
        agg = p0_b[0] + p1_b[0]
        deg = d0_b[0, :, :1] + d1_b[0, :, :1]
        msg = jnp.dot(agg, w1t_b[...],
                      preferred_element_type=jnp.float32) + deg * b1_b[...]
        lap = (deg * out_b[...] - msg) / jnp.maximum(deg, 1.0)
        o_b[...] = jnp.maximum(lap + skip_b[...], 0.0)

    full = lambda shape: pl.BlockSpec(shape, lambda i: tuple(0 for _ in shape))
    return pl.pallas_call(
        body,
        grid=(n // br,),
        in_specs=[
            pl.BlockSpec((br, d), lambda i: (i, 0)),
            pl.BlockSpec((br, d), lambda i: (i, 0)),
            pl.BlockSpec((1, br, d), lambda i: (0, i, 0)),
            pl.BlockSpec((1, br, d), lambda i: (1, i, 0)),
            pl.BlockSpec((1, br, DW), lambda i: (0, i, 0)),
            pl.BlockSpec((1, br, DW), lambda i: (1, i, 0)),
            full((d, d)), full((1, d)),
        ],
        out_specs=pl.BlockSpec((br, d), lambda i: (i, 0)),
        out_shape=jax.ShapeDtypeStruct((n, d), jnp.float32),
    )(out, skip, partials, partials, degs, degs, w1t, b1)


def kernel(data, merge, structure, W_lin, b_lin, W_tr, b_tr):
    n, d = data.shape
    n_pad = ((n + 127) // 128) * 128

    w1t = W_lin.T
    w2t = W_tr.T
    out, skip = _mm(data, merge, w1t, b_lin[None, :], w2t, b_tr[None, :], n, d)
    partials, degs = _sc_segment_sum(data, structure, n_pad, d)
    return _combine(out, skip, partials, degs, w1t, b_lin[None, :], n, d)


# speedup vs baseline: 1.0400x; 1.0113x over previous
import functools

import jax
import jax.numpy as jnp
from jax import lax
from jax.experimental import pallas as pl
from jax.experimental.pallas import tpu as pltpu
from jax.experimental.pallas import tpu_sc as plsc

NC = 2
NS = 16
CH = 128
TL = 16
DW = 16
CPB = 6
IBW = CPB * CH


def _sc_segment_sum(data, structure, n_pad, d):
    n, e = data.shape[0], structure.shape[1]
    del n
    edges_per_tile = e // (NC * NS)
    chunks = edges_per_tile // CH
    assert chunks * CH + TL == edges_per_tile and chunks % 2 == 0
    nblk = chunks // CPB
    assert nblk * CPB == chunks and nblk % 2 == 1
    ppb = CPB // 2
    rows_per_tile = n_pad // NS
    mesh = plsc.VectorSubcoreMesh(core_axis_name="c", subcore_axis_name="s")

    @functools.partial(
        pl.kernel,
        mesh=mesh,
        compiler_params=pltpu.CompilerParams(use_tc_tiling_on_sc=False),
        out_type=[
            jax.ShapeDtypeStruct((NC, n_pad, d), jnp.float32),
            jax.ShapeDtypeStruct((NC, n_pad, DW), jnp.float32),
        ],
        scratch_types=[
            pltpu.VMEM((2, IBW), jnp.int32),
            pltpu.VMEM((2, IBW), jnp.int32),
            pltpu.VMEM((CH, d), jnp.float32),
            pltpu.VMEM((CH, d), jnp.float32),
            pltpu.VMEM((CH, DW), jnp.float32),
            pltpu.VMEM_SHARED((n_pad, d), jnp.float32),
            pltpu.VMEM_SHARED((n_pad, DW), jnp.float32),
            pltpu.SemaphoreType.DMA,
            pltpu.SemaphoreType.DMA,
            pltpu.SemaphoreType.DMA,
            pltpu.SemaphoreType.DMA,
            pltpu.SemaphoreType.DMA,
        ],
    )
    def sc_kernel(data_hbm, struct_hbm, out_hbm, deg_hbm,
                  idxa, idxb, rows0, rows1, ones_v,
                  out_sh, deg_sh, sem0, sem1, semi, ssem0, ssem1):
        cid = lax.axis_index("c")
        sid = lax.axis_index("s")
        wid = cid * NS + sid
        ebase = wid * edges_per_tile
        rbase = sid * rows_per_tile

        def blk_refs(buf, b):
            return [(struct_hbm.at[r, pl.ds(ebase + b * IBW, IBW)], buf.at[r])
                    for r in range(2)]

        def fetch_blk(buf, b):
            for s, t in blk_refs(buf, b):
                pltpu.async_copy(s, t, semi)

        def wait_blk(buf, b):
            for s, t in blk_refs(buf, b):
                pltpu.make_async_copy(s, t, semi).wait()

        fetch_blk(idxa, 0)

        @pl.loop(0, CH)
        def _(i):
            ones_v.at[pl.ds(i, 1), :][...] = jnp.ones((1, DW), jnp.float32)

            @pl.loop(0, d, step=DW)
            def _(c):
                rows0.at[pl.ds(i, 1), pl.ds(c, DW)][...] = (
                    jnp.zeros((1, DW), jnp.float32))

        zrow = rows0.at[pl.ds(0, CH), pl.ds(0, DW)]
        zrow_t = rows0.at[pl.ds(0, 120), pl.ds(0, DW)]

        @pl.loop(0, 4)
        def _(i):
            pltpu.async_copy(rows0, out_sh.at[pl.ds(rbase + i * CH, CH)], semi)
            pltpu.async_copy(zrow, deg_sh.at[pl.ds(rbase + i * CH, CH)], semi)
        pltpu.async_copy(rows0.at[pl.ds(0, 120)],
                         out_sh.at[pl.ds(rbase + 4 * CH, 120)], semi)
        pltpu.async_copy(zrow_t, deg_sh.at[pl.ds(rbase + 4 * CH, 120)], semi)

        @pl.loop(0, 4)
        def _(i):
            pltpu.make_async_copy(
                rows0, out_sh.at[pl.ds(rbase + i * CH, CH)], semi).wait()
            pltpu.make_async_copy(
                zrow, deg_sh.at[pl.ds(rbase + i * CH, CH)], semi).wait()
        pltpu.make_async_copy(
            rows0.at[pl.ds(0, 120)],
            out_sh.at[pl.ds(rbase + 4 * CH, 120)], semi).wait()
        pltpu.make_async_copy(
            zrow_t, deg_sh.at[pl.ds(rbase + 4 * CH, 120)], semi).wait()
        wait_blk(idxa, 0)
        plsc.subcore_barrier()

        def gather(buf, l, rows, sem):
            pltpu.async_copy(data_hbm.at[buf.at[0, pl.ds(l * CH, CH)]],
                             rows, sem)

        def wait_gather(buf, l, rows, sem):
            pltpu.make_async_copy(data_hbm.at[buf.at[0, pl.ds(l * CH, CH)]],
                                  rows, sem).wait()

        def scatter(buf, l, rows, ssem):
            pltpu.async_copy(rows, out_sh.at[buf.at[1, pl.ds(l * CH, CH)]],
                             ssem, add=True)
            pltpu.async_copy(ones_v, deg_sh.at[buf.at[1, pl.ds(l * CH, CH)]],
                             ssem, add=True)

        def wait_scatter(buf, l, rows, ssem):
            pltpu.make_async_copy(
                rows, out_sh.at[buf.at[1, pl.ds(l * CH, CH)]], ssem).wait()
            pltpu.make_async_copy(
                ones_v, deg_sh.at[buf.at[1, pl.ds(l * CH, CH)]], ssem).wait()

        gather(idxa, 0, rows0, sem0)

        def do_block(b, cur, nxt, last):
            @pl.loop(0, ppb)
            def _(p):
                l0 = 2 * p
                l1 = l0 + 1
                wait_gather(cur, l0, rows0, sem0)

                @pl.when(p == 0)
                def _():
                    @pl.when(b > 0)
                    def _():
                        wait_scatter(nxt, CPB - 1, rows1, ssem1)
                    if not last:
                        fetch_blk(nxt, b + 1)

                @pl.when(p > 0)
                def _():
                    wait_scatter(cur, l0 - 1, rows1, ssem1)

                gather(cur, l1, rows1, sem1)
                scatter(cur, l0, rows0, ssem0)

                wait_gather(cur, l1, rows1, sem1)
                wait_scatter(cur, l0, rows0, ssem0)

                @pl.when(p < ppb - 1)
                def _():
                    gather(cur, l0 + 2, rows0, sem0)

                if not last:
                    @pl.when(p == ppb - 1)
                    def _():
                        wait_blk(nxt, b + 1)
                        gather(nxt, 0, rows0, sem0)

                scatter(cur, l1, rows1, ssem1)

        @pl.loop(0, nblk // 2)
        def _(sb):
            do_block(2 * sb, idxa, idxb, last=False)
            do_block(2 * sb + 1, idxb, idxa, last=False)
        do_block(nblk - 1, idxa, idxb, last=True)

        toff = ebase + chunks * CH
        pltpu.async_copy(struct_hbm.at[0, pl.ds(toff, TL)],
                         idxb.at[0, pl.ds(0, TL)], semi)
        pltpu.async_copy(struct_hbm.at[1, pl.ds(toff, TL)],
                         idxb.at[1, pl.ds(0, TL)], semi)
        pltpu.make_async_copy(struct_hbm.at[0, pl.ds(toff, TL)],
                              idxb.at[0, pl.ds(0, TL)], semi).wait()
        pltpu.make_async_copy(struct_hbm.at[1, pl.ds(toff, TL)],
                              idxb.at[1, pl.ds(0, TL)], semi).wait()
        wait_scatter(idxa, CPB - 1, rows1, ssem1)
        pltpu.async_copy(data_hbm.at[idxb.at[0, pl.ds(0, TL)]],
                         rows1.at[pl.ds(0, TL)], sem0).wait()
        pltpu.sync_copy(rows1.at[pl.ds(0, TL)],
                        out_sh.at[idxb.at[1, pl.ds(0, TL)]], add=True)
        pltpu.sync_copy(ones_v.at[pl.ds(0, TL)],
                        deg_sh.at[idxb.at[1, pl.ds(0, TL)]], add=True)

        plsc.subcore_barrier()
        pltpu.sync_copy(out_sh.at[pl.ds(rbase, rows_per_tile)],
                        out_hbm.at[cid].at[pl.ds(rbase, rows_per_tile)])
        pltpu.sync_copy(deg_sh.at[pl.ds(rbase, rows_per_tile)],
                        deg_hbm.at[cid].at[pl.ds(rbase, rows_per_tile)])

    return sc_kernel(data, structure)


def _mm(data, merge, w1t, b1, w2t, b2, n, d):
    br = 2000

    def body(data_b, merge_b, w1t_b, b1_b, w2t_b, b2_b, o_b, s_b):
        o_b[...] = jnp.dot(data_b[...], w1t_b[...],
                           preferred_element_type=jnp.float32) + b1_b[...]
        s_b[...] = jnp.dot(merge_b[...], w2t_b[...],
                           preferred_element_type=jnp.float32) + b2_b[...]

    full = lambda shape: pl.BlockSpec(shape, lambda i: tuple(0 for _ in shape))
    return pl.pallas_call(
        body,
        grid=(n // br,),
        in_specs=[
            pl.BlockSpec((br, d), lambda i: (i, 0)),
            pl.BlockSpec((br, d), lambda i: (i, 0)),
            full((d, d)), full((1, d)), full((d, d)), full((1, d)),
        ],
        out_specs=[pl.BlockSpec((br, d), lambda i: (i, 0)),
                   pl.BlockSpec((br, d), lambda i: (i, 0))],
        out_shape=[jax.ShapeDtypeStruct((n, d), jnp.float32),
                   jax.ShapeDtypeStruct((n, d), jnp.float32)],
    )(data, merge, w1t, b1, w2t, b2)


def _combine(out, skip, partials, degs, w1t, b1, n, d):
    br = 2000

    def body(out_b, skip_b, p0_b, p1_b, d0_b, d1_b, w1t_b, b1_b, o_b):
        agg = p0_b[0] + p1_b[0]
        deg = d0_b[0, :, :1] + d1_b[0, :, :1]
        msg = jnp.dot(agg, w1t_b[...],
                      preferred_element_type=jnp.float32) + deg * b1_b[...]
        lap = (deg * out_b[...] - msg) / jnp.maximum(deg, 1.0)
        o_b[...] = jnp.maximum(lap + skip_b[...], 0.0)

    full = lambda shape: pl.BlockSpec(shape, lambda i: tuple(0 for _ in shape))
    return pl.pallas_call(
        body,
        grid=(n // br,),
        in_specs=[
            pl.BlockSpec((br, d), lambda i: (i, 0)),
            pl.BlockSpec((br, d), lambda i: (i, 0)),
            pl.BlockSpec((1, br, d), lambda i: (0, i, 0)),
            pl.BlockSpec((1, br, d), lambda i: (1, i, 0)),
            pl.BlockSpec((1, br, DW), lambda i: (0, i, 0)),
            pl.BlockSpec((1, br, DW), lambda i: (1, i, 0)),
            full((d, d)), full((1, d)),
        ],
        out_specs=pl.BlockSpec((br, d), lambda i: (i, 0)),
        out_shape=jax.ShapeDtypeStruct((n, d), jnp.float32),
    )(out, skip, partials, partials, degs, degs, w1t, b1)


def kernel(data, merge, structure, W_lin, b_lin, W_tr, b_tr):
    n, d = data.shape
    n_pad = ((n + 127) // 128) * 128

    w1t = W_lin.T
    w2t = W_tr.T
    out, skip = _mm(data, merge, w1t, b_lin[None, :], w2t, b_tr[None, :], n, d)
    partials, degs = _sc_segment_sum(data, structure, n_pad, d)
    return _combine(out, skip, partials, degs, w1t, b_lin[None, :], n, d)
